# baseline (speedup 1.0000x reference)
import functools

import jax
import jax.numpy as jnp
from jax import lax
from jax.experimental import pallas as pl
from jax.experimental.pallas import tpu as pltpu
from jax.experimental.pallas import tpu_sc as plsc

_NC = 2
_NS = 16
_NW = _NC * _NS

_B = 4096 * 200
_D = 64
_IW = 128

_BPW = _B // _NW
_C = 512
_K = _C // _IW
_NCHUNK = _BPW // _C


def _gather_body(table_hbm, idx_hbm, out_hbm, idx_v, rows_v, sem_g):
    wid = lax.axis_index("s") * _NC + lax.axis_index("c")
    row0 = wid * (_BPW // _IW)

    def chunk(g, _):
        pltpu.sync_copy(idx_hbm.at[pl.ds(row0 + g * _K, _K)], idx_v)
        for j in range(_K):
            pltpu.async_copy(
                table_hbm.at[idx_v.at[j]],
                rows_v.at[pl.ds(j * _IW, _IW)],
                sem_g,
            )
        for j in range(_K):
            pltpu.make_async_copy(
                table_hbm.at[idx_v.at[j]],
                rows_v.at[pl.ds(j * _IW, _IW)],
                sem_g,
            ).wait()
        base = wid * _BPW + g * _C
        pltpu.sync_copy(rows_v, out_hbm.at[pl.ds(base, _C)])
        return _

    lax.fori_loop(0, _NCHUNK, chunk, 0)


@jax.jit
def _embedding_lookup(input_, weight):
    idx2d = input_.reshape(_B // _IW, _IW).astype(jnp.int32)
    mesh = plsc.VectorSubcoreMesh(core_axis_name="c", subcore_axis_name="s")
    out = pl.kernel(
        _gather_body,
        out_type=jax.ShapeDtypeStruct((_B, _D), jnp.float32),
        mesh=mesh,
        scratch_types=[
            pltpu.VMEM((_K, _IW), jnp.int32),
            pltpu.VMEM((_C, _D), jnp.float32),
            pltpu.SemaphoreType.DMA,
        ],
        compiler_params=pltpu.CompilerParams(use_tc_tiling_on_sc=False),
    )(weight, idx2d)
    return out.reshape(input_.shape[0], input_.shape[1], _D)


def kernel(input_, weight):
    return _embedding_lookup(input_, weight)


# speedup vs baseline: 1.0456x; 1.0456x over previous
import functools

import jax
import jax.numpy as jnp
from jax import lax
from jax.experimental import pallas as pl
from jax.experimental.pallas import tpu as pltpu
from jax.experimental.pallas import tpu_sc as plsc

_NC = 2
_NS = 16
_NW = _NC * _NS

_B = 4096 * 200
_D = 64
_IW = 128

_BPW = _B // _NW
_KALL = _BPW // _IW
_C = 512
_K = _C // _IW
_NCHUNK = _BPW // _C


def _gather_body(table_hbm, idx_hbm, out_hbm, idx_v, rows_v, sem_g, sem_o):
    wid = lax.axis_index("s") * _NC + lax.axis_index("c")
    row0 = wid * _KALL
    base0 = wid * _BPW

    pltpu.sync_copy(idx_hbm.at[pl.ds(row0, _KALL)], idx_v)

    def fire_gathers(g, buf):
        for j in range(_K):
            pltpu.async_copy(
                table_hbm.at[idx_v.at[g * _K + j]],
                rows_v.at[pl.ds(buf * _C + j * _IW, _IW)],
                sem_g,
            )

    def drain_gathers(g, buf):
        for j in range(_K):
            pltpu.make_async_copy(
                table_hbm.at[idx_v.at[g * _K + j]],
                rows_v.at[pl.ds(buf * _C + j * _IW, _IW)],
                sem_g,
            ).wait()

    def store(g, buf):
        pltpu.async_copy(
            rows_v.at[pl.ds(buf * _C, _C)],
            out_hbm.at[pl.ds(base0 + g * _C, _C)],
            sem_o,
        )

    def wait_store(g, buf):
        pltpu.make_async_copy(
            rows_v.at[pl.ds(buf * _C, _C)],
            out_hbm.at[pl.ds(base0 + g * _C, _C)],
            sem_o,
        ).wait()

    fire_gathers(0, 0)

    def chunk_pair(gg, _):
        for b in range(2):
            g = gg * 2 + b
            nb = 1 - b

            @pl.when(g < _NCHUNK - 1)
            def _fill_next():
                @pl.when(g >= 1)
                def _free_buf():
                    wait_store(g - 1, nb)

                fire_gathers(g + 1, nb)

            drain_gathers(g, b)
            store(g, b)
        return _

    lax.fori_loop(0, _NCHUNK // 2, chunk_pair, 0)
    wait_store(_NCHUNK - 2, 0)
    wait_store(_NCHUNK - 1, 1)


@jax.jit
def _embedding_lookup(input_, weight):
    idx2d = input_.reshape(_B // _IW, _IW).astype(jnp.int32)
    mesh = plsc.VectorSubcoreMesh(core_axis_name="c", subcore_axis_name="s")
    out = pl.kernel(
        _gather_body,
        out_type=jax.ShapeDtypeStruct((_B, _D), jnp.float32),
        mesh=mesh,
        scratch_types=[
            pltpu.VMEM((_KALL, _IW), jnp.int32),
            pltpu.VMEM((2 * _C, _D), jnp.float32),
            pltpu.SemaphoreType.DMA,
            pltpu.SemaphoreType.DMA,
        ],
        compiler_params=pltpu.CompilerParams(use_tc_tiling_on_sc=False),
    )(weight, idx2d)
    return out.reshape(input_.shape[0], input_.shape[1], _D)


def kernel(input_, weight):
    return _embedding_lookup(input_, weight)
